# baseline (speedup 1.0000x reference)
import functools
import math

import jax
import jax.numpy as jnp
from jax import lax
from jax.experimental import pallas as pl
from jax.experimental.pallas import tpu as pltpu
from jax.experimental.pallas import tpu_sc as plsc

_info = plsc.get_sparse_core_info()
_NC, _NS, _L = _info.num_cores, _info.num_subcores, _info.num_lanes
_NW = _NC * _NS
_PAD = 1
_DIAG_SKIP_COMPUTE = False


def _make_pack(V: int, D: int, scale: float):
  W = 2 * D
  n_full = V // W
  n_main = (n_full // _NW) * _NW
  rem_words = V - n_full * W
  mesh = plsc.VectorSubcoreMesh(core_axis_name="c", subcore_axis_name="s")

  @functools.partial(
      pl.kernel,
      mesh=mesh,
      out_type=jax.ShapeDtypeStruct((V // 2, 2 * D), jnp.float32),
      compiler_params=pltpu.CompilerParams(needs_layout_passes=False),
      scratch_types=[
          [pltpu.VMEM((D, W + _PAD), jnp.float32)] * 2,
          [pltpu.VMEM((D, W), jnp.float32)] * 2,
          [pltpu.SemaphoreType.DMA] * 2,
          [pltpu.SemaphoreType.DMA] * 2,
      ],
  )
  def pack_kernel(tt_hbm, tail_hbm, out_hbm, vbuf, obuf, isem, osem):
    wid = lax.axis_index("s") * _NC + lax.axis_index("c")

    def fire_in(g, b):
      off = pl.multiple_of(g * W, W)
      pltpu.async_copy(tt_hbm.at[:, pl.ds(off, W)],
                       vbuf[b].at[:, pl.ds(0, W)], isem[b])

    def transpose_block(vb, ob, nrows, col_off=0):
      rows0 = [k * _L + lax.iota(jnp.int32, _L) for k in range(D // _L)]

      @plsc.parallel_loop(0, nrows, unroll=4)
      def row_body(j):
        for p in (0, 1):
          col = jnp.full((_L,), 2 * j + p + col_off, jnp.int32)
          for k in range(D // _L):
            vals = plsc.load_gather(vb, [rows0[k], col])
            ob[j, pl.ds(p * D + k * _L, _L)] = vals * scale

    for b in (0, 1):
      fire_in(wid + b * _NW, b)

    def outer_body(ko, carry):
      for b in (0, 1):
        k = 2 * ko + b
        g = wid + k * _NW
        off = pl.multiple_of(g * W, W)
        pltpu.make_async_copy(tt_hbm.at[:, pl.ds(off, W)],
                              vbuf[b].at[:, pl.ds(0, W)], isem[b]).wait()
        @pl.when(ko > 0)
        def _():
          pltpu.make_async_copy(obuf[b], out_hbm.at[pl.ds(0, D)],
                                osem[b]).wait()

        if not _DIAG_SKIP_COMPUTE:
          transpose_block(vbuf[b], obuf[b], D)
        roff = pl.multiple_of(g * D, D)
        pltpu.async_copy(obuf[b], out_hbm.at[pl.ds(roff, D)], osem[b])

        @pl.when(k + 2 < n_main // _NW)
        def _():
          fire_in(g + 2 * _NW, b)

      return carry

    lax.fori_loop(0, (n_main // _NW) // 2, outer_body, 0)
    for b in (0, 1):
      pltpu.make_async_copy(obuf[b], out_hbm.at[pl.ds(0, D)], osem[b]).wait()

    @pl.when(wid < n_full - n_main)
    def _():
      g = n_main + wid
      off = pl.multiple_of(g * W, W)
      pltpu.sync_copy(tt_hbm.at[:, pl.ds(off, W)],
                      vbuf[0].at[:, pl.ds(0, W)])
      transpose_block(vbuf[0], obuf[0], D)
      roff = pl.multiple_of(g * D, D)
      pltpu.sync_copy(obuf[0], out_hbm.at[pl.ds(roff, D)])

    if rem_words:
      @pl.when(wid == n_full - n_main)
      def _():
        nt = rem_words // 2
        pltpu.sync_copy(tail_hbm, obuf[0].at[pl.ds(0, nt)])
        pltpu.sync_copy(obuf[0].at[pl.ds(0, nt)],
                        out_hbm.at[pl.ds((V - rem_words) // 2, nt)])

  return pack_kernel


def _make_gather(BSZ: int, H: int, VP: int, D: int):
  NB = BSZ // _NW
  W = 2 * D
  n_groups = NB // _L
  mesh = plsc.VectorSubcoreMesh(core_axis_name="c", subcore_axis_name="s")

  @functools.partial(
      pl.kernel,
      mesh=mesh,
      out_type=jax.ShapeDtypeStruct((H, D, BSZ), jnp.float32),
      compiler_params=pltpu.CompilerParams(needs_layout_passes=False),
      scratch_types=[
          pltpu.VMEM((H, NB), jnp.int32),
          pltpu.VMEM((2, NB), jnp.int32),
          [pltpu.VMEM((NB, W + _PAD), jnp.float32)] * 2,
          [pltpu.VMEM((D, NB + _PAD), jnp.float32)] * 2,
          [pltpu.SemaphoreType.DMA] * 2,
          [pltpu.SemaphoreType.DMA] * 2,
      ],
  )
  def gather_kernel(packed_hbm, seqt_hbm, out_hbm, idx_all, half_buf, gbuf,
                    tbuf, gsem, ssem):
    wid = lax.axis_index("s") * _NC + lax.axis_index("c")
    b0 = pl.multiple_of(wid * NB, NB)

    pltpu.sync_copy(seqt_hbm.at[:, pl.ds(b0, NB)], idx_all)

    def fire_gather(h, b):
      for jg in range(n_groups):
        sl = pl.ds(jg * _L, _L)
        half_buf[b, sl] = idx_all[h, sl] >> 1
      pltpu.async_copy(packed_hbm.at[half_buf.at[b]],
                       gbuf[b].at[:, pl.ds(0, W)], gsem[b])

    for b in (0, 1):
      fire_gather(b, b)

    def outer_body(go, carry):
      for b in (0, 1):
        h = 2 * go + b
        pltpu.make_async_copy(packed_hbm.at[half_buf.at[b]],
                              gbuf[b].at[:, pl.ds(0, W)], gsem[b]).wait()
        @pl.when(go > 0)
        def _():
          pltpu.make_async_copy(tbuf[b].at[:, pl.ds(0, NB)],
                                out_hbm.at[0, :, pl.ds(b0, NB)],
                                ssem[b]).wait()

        if not _DIAG_SKIP_COMPUTE:
          @plsc.parallel_loop(0, n_groups, unroll=4)
          def comp(jg):
            j0 = jg * _L
            sl = pl.ds(j0, _L)
            jids = j0 + lax.iota(jnp.int32, _L)
            colbase = (idx_all[h, sl] & 1) * D
            for c in range(D):
              vals = plsc.load_gather(gbuf[b], [jids, colbase + c])
              tbuf[b][c, sl] = vals

        pltpu.async_copy(tbuf[b].at[:, pl.ds(0, NB)],
                         out_hbm.at[h, :, pl.ds(b0, NB)], ssem[b])

        @pl.when(h + 2 < H)
        def _():
          fire_gather(h + 2, b)

      return carry

    lax.fori_loop(0, H // 2, outer_body, 0)
    for b in (0, 1):
      pltpu.make_async_copy(tbuf[b].at[:, pl.ds(0, NB)],
                            out_hbm.at[0, :, pl.ds(b0, NB)], ssem[b]).wait()

  return gather_kernel


def kernel(seq, table):
  bsz, hist = seq.shape
  V, D = table.shape
  scale = math.sqrt(D)
  rem = V % (2 * D)
  tail = (table[V - rem:] * scale).reshape(rem // 2, 2 * D)
  packed = _make_pack(V, D, scale)(table.T, tail)
  out3 = _make_gather(bsz, hist, V // 2, D)(packed, seq.T)
  return jnp.transpose(out3, (2, 0, 1))


# speedup vs baseline: 1.3243x; 1.3243x over previous
import functools
import math

import jax
import jax.numpy as jnp
from jax import lax
from jax.experimental import pallas as pl
from jax.experimental.pallas import tpu as pltpu
from jax.experimental.pallas import tpu_sc as plsc

_info = plsc.get_sparse_core_info()
_NC, _NS, _L = _info.num_cores, _info.num_subcores, _info.num_lanes
_NW = _NC * _NS


def _make_gather(B: int, V: int, D: int, C: int):
  assert B % (_NW * C) == 0 and C % 8 == 0 and D % _L == 0
  scale = math.sqrt(D)
  b_per_w = B // _NW
  n_chunks = b_per_w // C
  mesh = plsc.VectorSubcoreMesh(core_axis_name="c", subcore_axis_name="s")

  @functools.partial(
      pl.kernel,
      mesh=mesh,
      out_type=jax.ShapeDtypeStruct((B, D), jnp.float32),
      compiler_params=pltpu.CompilerParams(use_tc_tiling_on_sc=False),
      scratch_types=[
          pltpu.VMEM((C,), jnp.int32),
          pltpu.VMEM((C, D), jnp.float32),
          pltpu.SemaphoreType.DMA,
      ],
  )
  def gather_kernel(table_hbm, idx_hbm, out_hbm, idx_v, rows_v, sem):
    wid = lax.axis_index("s") * _NC + lax.axis_index("c")
    base = wid * b_per_w

    def chunk_body(g, carry):
      off = base + g * C
      pltpu.sync_copy(idx_hbm.at[pl.ds(off, C)], idx_v)
      pltpu.async_copy(table_hbm.at[idx_v], rows_v, sem).wait()

      @plsc.parallel_loop(0, C, unroll=4)
      def scale_row(j):
        for k in range(D // _L):
          sl = pl.ds(k * _L, _L)
          rows_v[j, sl] = rows_v[j, sl] * scale

      pltpu.sync_copy(rows_v, out_hbm.at[pl.ds(off, C)])
      return carry

    lax.fori_loop(0, n_chunks, chunk_body, 0)

  return gather_kernel


def kernel(seq, table):
  bsz, hist = seq.shape
  B = bsz * hist
  V, D = table.shape
  idx = seq.reshape(B)
  out = _make_gather(B, V, D, C=1600)(table, idx)
  return out.reshape(bsz, hist, D)


# speedup vs baseline: 1.3879x; 1.0481x over previous
import functools
import math

import jax
import jax.numpy as jnp
from jax import lax
from jax.experimental import pallas as pl
from jax.experimental.pallas import tpu as pltpu
from jax.experimental.pallas import tpu_sc as plsc

_info = plsc.get_sparse_core_info()
_NC, _NS, _L = _info.num_cores, _info.num_subcores, _info.num_lanes
_NW = _NC * _NS


def _make_gather(B: int, V: int, D: int, C: int):
  assert B % (_NW * C) == 0 and C % 8 == 0 and D % _L == 0
  scale = math.sqrt(D)
  b_per_w = B // _NW
  n_chunks = b_per_w // C
  mesh = plsc.VectorSubcoreMesh(core_axis_name="c", subcore_axis_name="s")

  @functools.partial(
      pl.kernel,
      mesh=mesh,
      out_type=jax.ShapeDtypeStruct((B, D), jnp.float32),
      compiler_params=pltpu.CompilerParams(use_tc_tiling_on_sc=False),
      scratch_types=[
          [pltpu.VMEM((C,), jnp.int32)] * 2,
          [pltpu.VMEM((C, D), jnp.float32)] * 2,
          [pltpu.SemaphoreType.DMA] * 2,
          [pltpu.SemaphoreType.DMA] * 2,
      ],
  )
  def gather_kernel(table_hbm, idx_hbm, out_hbm, idx_v, rows_v, gsem, ssem):
    wid = lax.axis_index("s") * _NC + lax.axis_index("c")
    base = wid * b_per_w

    def fire(g, b):
      off = base + g * C
      pltpu.sync_copy(idx_hbm.at[pl.ds(off, C)], idx_v[b])
      pltpu.async_copy(table_hbm.at[idx_v[b]], rows_v[b], gsem[b])

    for b in (0, 1):
      fire(b, b)

    def outer_body(go, carry):
      for b in (0, 1):
        g = 2 * go + b
        off = base + g * C
        pltpu.make_async_copy(table_hbm.at[idx_v[b]], rows_v[b],
                              gsem[b]).wait()

        @plsc.parallel_loop(0, C, unroll=4)
        def scale_row(j):
          for k in range(D // _L):
            sl = pl.ds(k * _L, _L)
            rows_v[b][j, sl] = rows_v[b][j, sl] * scale

        pltpu.async_copy(rows_v[b], out_hbm.at[pl.ds(off, C)], ssem[b])

        @pl.when(g + 2 < n_chunks)
        def _():
          pltpu.make_async_copy(rows_v[b], out_hbm.at[pl.ds(base, C)],
                                ssem[b]).wait()
          fire(g + 2, b)

      return carry

    lax.fori_loop(0, n_chunks // 2, outer_body, 0)
    for b in (0, 1):
      pltpu.make_async_copy(rows_v[b], out_hbm.at[pl.ds(base, C)],
                            ssem[b]).wait()

  return gather_kernel


def kernel(seq, table):
  bsz, hist = seq.shape
  B = bsz * hist
  V, D = table.shape
  idx = seq.reshape(B)
  out = _make_gather(B, V, D, C=800)(table, idx)
  return out.reshape(bsz, hist, D)
